# Initial kernel scaffold; baseline (speedup 1.0000x reference)
import jax
import jax.numpy as jnp
from jax.experimental import pallas as pl


def kernel(x, edge_index, W1, b1, W2, b2, W3, b3, Wl, bl):
    raise NotImplementedError("write your pallas kernel here")


# speedup vs baseline: 14.7373x; 14.7373x over previous
import functools

import jax
import jax.numpy as jnp
from jax import lax
from jax.experimental import pallas as pl
from jax.experimental.pallas import tpu as pltpu
from jax.experimental.pallas import tpu_sc as plsc

N = 10000
NPAD = 10240
D = 128
E = 320000
EALL = E + N
EPAD = 330240
NC = 2
NS = 16
NT = NC * NS
EPT = EPAD // NT
EPC = EPAD // NS
NPS = NPAD // NS
SLW = NPS * 4
NPAD4 = NPAD * 4
EPT4 = EPT * 4
CHUNK = EPT4 // 2

_f32 = jnp.float32
_i32 = jnp.int32


def _mm_body(x_ref, w_ref, o_ref):
    o_ref[...] = jnp.dot(x_ref[...], w_ref[...], preferred_element_type=_f32)


def _tc_matmul(xp, w1):
    return pl.pallas_call(
        _mm_body,
        grid=(5,),
        in_specs=[
            pl.BlockSpec((2048, D), lambda i: (i, 0)),
            pl.BlockSpec((D, 4), lambda i: (0, 0)),
        ],
        out_specs=pl.BlockSpec((2048, 4), lambda i: (i, 0)),
        out_shape=jax.ShapeDtypeStruct((NPAD, 4), _f32),
    )(xp, w1)


def _fill(ref, nwords, value):
    v = jnp.full((16,), value, _f32)

    def body(j, c):
        ref[pl.ds(j * 16, 16)] = v
        return c

    lax.fori_loop(0, nwords // 16, body, 0)


def _rsqrt_newton(d):
    d = jnp.maximum(d, 0.25)
    i = lax.bitcast_convert_type(d, _i32)
    i = 0x5F3759DF - lax.shift_right_logical(i, 1)
    y = lax.bitcast_convert_type(i, _f32)
    for _ in range(3):
        y = y * (1.5 - 0.5 * d * y * y)
    return y


def _tanh16(v):
    vm = jnp.minimum(jnp.maximum(v, -15.0), 15.0)
    e = jnp.exp(2.0 * vm)
    return 1.0 - 2.0 / (e + 1.0)


def _edge_phase(src4_hbm, dst4_hbm, gs_sh, acc_sh, ia_v, ib_v, up_v, wid):
    def body(c, k):
        base = wid * EPT4 + c * CHUNK
        pltpu.sync_copy(src4_hbm.at[pl.ds(base, CHUNK)], ia_v)
        pltpu.sync_copy(dst4_hbm.at[pl.ds(base, CHUNK)], ib_v)
        pltpu.sync_copy(gs_sh.at[ia_v], up_v)
        pltpu.sync_copy(up_v, acc_sh.at[ib_v], add=True)
        return k

    lax.fori_loop(0, EPT4 // CHUNK, body, 0)


_sc_mesh = plsc.VectorSubcoreMesh(core_axis_name="c", subcore_axis_name="s")


@functools.partial(
    pl.kernel,
    out_type=(
        jax.ShapeDtypeStruct((NPAD,), _f32),
        jax.ShapeDtypeStruct((NC * NPAD4,), _f32),
    ),
    mesh=_sc_mesh,
    compiler_params=pltpu.CompilerParams(needs_layout_passes=False),
    scratch_types=[
        pltpu.VMEM_SHARED((NPAD,), _f32),
        pltpu.VMEM_SHARED((NPAD4,), _f32),
        pltpu.VMEM_SHARED((NPAD4,), _f32),
        pltpu.VMEM((CHUNK,), _i32),
        pltpu.VMEM((CHUNK,), _i32),
        pltpu.VMEM((CHUNK,), _f32),
        pltpu.VMEM((NPAD,), _f32),
        pltpu.VMEM((SLW,), _f32),
    ],
)
def _sc_prep(dst_hbm, src4_hbm, dst4_hbm, g1_hbm, dinv_hbm, pout_hbm,
             deg_sh, gs_sh, acc_sh, ia_v, ib_v, up_v, dinv_v, sl_v):
    cid = lax.axis_index("c")
    sid = lax.axis_index("s")
    wid = sid * NC + cid
    iota = lax.iota(_i32, 16)
    nodeoff = lax.shift_right_logical(iota, 2)

    _fill(sl_v, SLW, 0.0)
    pltpu.sync_copy(sl_v.at[pl.ds(0, NPS)], deg_sh.at[pl.ds(sid * NPS, NPS)])
    pltpu.sync_copy(sl_v, acc_sh.at[pl.ds(sid * SLW, SLW)])
    _fill(up_v, CHUNK, 1.0)
    plsc.subcore_barrier()

    pltpu.sync_copy(dst_hbm.at[pl.ds(sid * EPC, EPC)], ia_v)
    pltpu.sync_copy(up_v, deg_sh.at[ia_v], add=True)
    plsc.subcore_barrier()

    pltpu.sync_copy(deg_sh, dinv_v)

    def rbody(j, c):
        dinv_v[pl.ds(j * 16, 16)] = _rsqrt_newton(dinv_v[pl.ds(j * 16, 16)])
        return c

    lax.fori_loop(0, NPAD // 16, rbody, 0)

    @pl.when(cid == 0)
    def _():
        pltpu.sync_copy(dinv_v.at[pl.ds(sid * NPS, NPS)],
                        dinv_hbm.at[pl.ds(sid * NPS, NPS)])

    pltpu.sync_copy(g1_hbm.at[pl.ds(sid * SLW, SLW)], sl_v)

    def gbody(j, c):
        v = sl_v[pl.ds(j * 16, 16)]
        dix = plsc.load_gather(dinv_v, [sid * NPS + j * 4 + nodeoff])
        sl_v[pl.ds(j * 16, 16)] = v * dix
        return c

    lax.fori_loop(0, SLW // 16, gbody, 0)
    pltpu.sync_copy(sl_v, gs_sh.at[pl.ds(sid * SLW, SLW)])
    plsc.subcore_barrier()

    _edge_phase(src4_hbm, dst4_hbm, gs_sh, acc_sh, ia_v, ib_v, up_v, wid)
    plsc.subcore_barrier()

    pltpu.sync_copy(acc_sh.at[pl.ds(sid * SLW, SLW)],
                    pout_hbm.at[pl.ds(cid * NPAD4 + sid * SLW, SLW)])


@functools.partial(
    pl.kernel,
    out_type=jax.ShapeDtypeStruct((NC * NPAD4,), _f32),
    mesh=_sc_mesh,
    compiler_params=pltpu.CompilerParams(needs_layout_passes=False),
    scratch_types=[
        pltpu.VMEM_SHARED((NPAD4,), _f32),
        pltpu.VMEM_SHARED((NPAD4,), _f32),
        pltpu.VMEM((CHUNK,), _i32),
        pltpu.VMEM((CHUNK,), _i32),
        pltpu.VMEM((CHUNK,), _f32),
        pltpu.VMEM((SLW,), _f32),
        pltpu.VMEM((SLW,), _f32),
        pltpu.VMEM((NPS,), _f32),
        pltpu.VMEM((SLW,), _f32),
        pltpu.VMEM((4, 16), _f32),
        pltpu.VMEM((16,), _f32),
        pltpu.VMEM((16,), _f32),
    ],
)
def _sc_layer(src4_hbm, dst4_hbm, pprev_hbm, dinv_hbm, bvec_hbm, wv_hbm,
              pout_hbm,
              gs_sh, acc_sh, ia_v, ib_v, up_v, pa_v, pb_v, dv_v, sl_v,
              wv_v, bv_v, tb_v):
    cid = lax.axis_index("c")
    sid = lax.axis_index("s")
    wid = sid * NC + cid
    iota = lax.iota(_i32, 16)
    nodeoff = lax.shift_right_logical(iota, 2)
    blk = jnp.bitwise_and(iota, 12)

    pltpu.sync_copy(wv_hbm, wv_v)
    pltpu.sync_copy(bvec_hbm, bv_v)
    pltpu.sync_copy(dinv_hbm.at[pl.ds(sid * NPS, NPS)], dv_v)
    pltpu.sync_copy(pprev_hbm.at[pl.ds(sid * SLW, SLW)], pa_v)
    pltpu.sync_copy(pprev_hbm.at[pl.ds(NPAD4 + sid * SLW, SLW)], pb_v)
    bvec = bv_v[...]
    w0 = wv_v[0, :]
    w1 = wv_v[1, :]
    w2 = wv_v[2, :]
    w3 = wv_v[3, :]

    def ebody(j, c):
        dix = plsc.load_gather(dv_v, [j * 4 + nodeoff])
        v = (pa_v[pl.ds(j * 16, 16)] + pb_v[pl.ds(j * 16, 16)]) * dix + bvec
        t = _tanh16(v)
        tb_v[...] = t
        acc = plsc.load_gather(tb_v, [blk]) * w0
        acc = acc + plsc.load_gather(tb_v, [blk + 1]) * w1
        acc = acc + plsc.load_gather(tb_v, [blk + 2]) * w2
        acc = acc + plsc.load_gather(tb_v, [blk + 3]) * w3
        sl_v[pl.ds(j * 16, 16)] = acc * dix
        return c

    lax.fori_loop(0, SLW // 16, ebody, 0)

    _fill(pa_v, SLW, 0.0)
    pltpu.sync_copy(pa_v, acc_sh.at[pl.ds(sid * SLW, SLW)])
    pltpu.sync_copy(sl_v, gs_sh.at[pl.ds(sid * SLW, SLW)])
    plsc.subcore_barrier()

    _edge_phase(src4_hbm, dst4_hbm, gs_sh, acc_sh, ia_v, ib_v, up_v, wid)
    plsc.subcore_barrier()

    pltpu.sync_copy(acc_sh.at[pl.ds(sid * SLW, SLW)],
                    pout_hbm.at[pl.ds(cid * NPAD4 + sid * SLW, SLW)])


def _fin_body(p_ref, dinv_ref, b3_ref, wl_ref, bl_ref, out_ref, h3_ref):
    h3 = (p_ref[0] + p_ref[1]) * dinv_ref[...] + b3_ref[...]
    h3_ref[...] = h3
    out_ref[...] = (
        jnp.dot(h3, wl_ref[...], preferred_element_type=_f32) + bl_ref[...]
    )


def _tc_final(p3, dinvb, b3p, wlp, blp):
    return pl.pallas_call(
        _fin_body,
        out_shape=(
            jax.ShapeDtypeStruct((NPAD, 4), _f32),
            jax.ShapeDtypeStruct((NPAD, 4), _f32),
        ),
    )(p3, dinvb, b3p, wlp, blp)


def kernel(x, edge_index, W1, b1, W2, b2, W3, b3, Wl, bl):
    xp = jnp.zeros((NPAD, D), _f32).at[:N].set(x)
    g1 = _tc_matmul(xp, W1)

    src = edge_index[0].astype(_i32)
    dst = edge_index[1].astype(_i32)
    loop = jnp.arange(N, dtype=_i32)
    padn = N + jnp.arange(EPAD - EALL, dtype=_i32) % (NPAD - N)
    src_all = jnp.concatenate([src, loop, padn])
    dst_all = jnp.concatenate([dst, loop, padn])
    four = jnp.arange(4, dtype=_i32)
    src4 = (src_all[:, None] * 4 + four[None, :]).reshape(-1)
    dst4 = (dst_all[:, None] * 4 + four[None, :]).reshape(-1)

    dinv, p1 = _sc_prep(dst_all, src4, dst4, g1.reshape(-1))

    b1t = jnp.tile(b1, 4)
    w2v = jnp.tile(W2, (1, 4))
    p2 = _sc_layer(src4, dst4, p1, dinv, b1t, w2v)

    b2t = jnp.tile(b2, 4)
    w3p = jnp.pad(W3, ((0, 0), (0, 2)))
    w3v = jnp.tile(w3p, (1, 4))
    p3 = _sc_layer(src4, dst4, p2, dinv, b2t, w3v)

    b3p = jnp.pad(b3, (0, 2)).reshape(1, 4)
    wlp = jnp.pad(Wl, ((0, 2), (0, 0)))
    blp = bl.reshape(1, 4)
    out_full, h3_full = _tc_final(
        p3.reshape(NC, NPAD, 4), dinv.reshape(NPAD, 1), b3p, wlp, blp
    )
    return out_full[:N], h3_full[:N, :2]
